# baseline (speedup 1.0000x reference)
import functools

import jax
import jax.numpy as jnp
from jax.experimental import pallas as pl
from jax.experimental.pallas import tpu as pltpu

K_TOP = 64


def _enc_kernel(x_ref, w_ref, b_ref, h_ref):
    acc = jax.lax.dot_general(
        x_ref[...], w_ref[...], (((1,), (1,)), ((), ())),
        preferred_element_type=jnp.float32)
    h_ref[...] = jnp.maximum(acc + b_ref[...].astype(jnp.float32), 0.0)


def _tau_kernel(h_ref, t_ref, *, k_top, n_iter):
    tblk, dh = h_ref.shape
    sl = 2048
    ns = dh // sl

    mx = jnp.zeros((tblk, 1), jnp.float32)
    for s in range(ns):
        mx = jnp.maximum(mx, jnp.max(h_ref[:, s * sl:(s + 1) * sl],
                                     axis=1, keepdims=True))
    hi = jax.lax.bitcast_convert_type(mx, jnp.int32)
    lo = jnp.zeros((tblk, 1), jnp.int32)

    def body(_, carry):
        lo, hi = carry
        mid = lo + ((hi - lo + 1) >> 1)
        cnt = jnp.zeros((tblk, 1), jnp.int32)
        for s in range(ns):
            hb = jax.lax.bitcast_convert_type(
                h_ref[:, s * sl:(s + 1) * sl], jnp.int32)
            cnt = cnt + jnp.sum(jnp.where(hb >= mid, 1, 0),
                                axis=1, keepdims=True)
        take = cnt >= k_top
        return (jnp.where(take, mid, lo), jnp.where(take, hi, mid - 1))

    lo, hi = jax.lax.fori_loop(0, n_iter, body, (lo, hi))
    t_ref[...] = jax.lax.bitcast_convert_type(lo, jnp.float32)


def _dec_kernel(h_ref, t_ref, w_ref, b_ref, o_ref, hm_ref):
    j = pl.program_id(1)
    hm = jnp.where(h_ref[...] >= t_ref[...], h_ref[...], 0.0)
    hm_ref[...] = hm
    acc = jax.lax.dot_general(
        hm.astype(jnp.bfloat16), w_ref[...],
        (((1,), (1,)), ((), ())),
        preferred_element_type=jnp.float32)

    @pl.when(j == 0)
    def _init():
        o_ref[...] = acc + b_ref[...].astype(jnp.float32)

    @pl.when(j > 0)
    def _accum():
        o_ref[...] += acc


def kernel(x, W_enc, b_enc, W_dec, b_dec):
    n_tok, d_in = x.shape
    d_hid = W_enc.shape[0]

    xb = x.astype(jnp.bfloat16)
    web = W_enc.astype(jnp.bfloat16)
    wdb = W_dec.astype(jnp.bfloat16)
    b_enc2 = b_enc.reshape(1, d_hid)
    b_dec2 = b_dec.reshape(1, d_in)

    ta, ha = min(1024, n_tok), 1024
    h_raw = pl.pallas_call(
        _enc_kernel,
        grid=(n_tok // ta, d_hid // ha),
        in_specs=[
            pl.BlockSpec((ta, d_in), lambda i, j: (i, 0)),
            pl.BlockSpec((ha, d_in), lambda i, j: (j, 0)),
            pl.BlockSpec((1, ha), lambda i, j: (0, j)),
        ],
        out_specs=pl.BlockSpec((ta, ha), lambda i, j: (i, j)),
        out_shape=jax.ShapeDtypeStruct((n_tok, d_hid), jnp.float32),
        compiler_params=pltpu.CompilerParams(
            dimension_semantics=("parallel", "arbitrary")),
    )(xb, web, b_enc2)

    tm = min(128, n_tok)
    tau = pl.pallas_call(
        functools.partial(_tau_kernel, k_top=K_TOP, n_iter=31),
        grid=(n_tok // tm,),
        in_specs=[pl.BlockSpec((tm, d_hid), lambda i: (i, 0))],
        out_specs=pl.BlockSpec((tm, 1), lambda i: (i, 0)),
        out_shape=jax.ShapeDtypeStruct((n_tok, 1), jnp.float32),
        compiler_params=pltpu.CompilerParams(
            dimension_semantics=("parallel",)),
    )(h_raw)

    tb, hb_ = min(512, n_tok), 1024
    recon, h = pl.pallas_call(
        _dec_kernel,
        grid=(n_tok // tb, d_hid // hb_),
        in_specs=[
            pl.BlockSpec((tb, hb_), lambda i, j: (i, j)),
            pl.BlockSpec((tb, 1), lambda i, j: (i, 0)),
            pl.BlockSpec((d_in, hb_), lambda i, j: (0, j)),
            pl.BlockSpec((1, d_in), lambda i, j: (0, 0)),
        ],
        out_specs=(
            pl.BlockSpec((tb, d_in), lambda i, j: (i, 0)),
            pl.BlockSpec((tb, hb_), lambda i, j: (i, j)),
        ),
        out_shape=(
            jax.ShapeDtypeStruct((n_tok, d_in), jnp.float32),
            jax.ShapeDtypeStruct((n_tok, d_hid), jnp.float32),
        ),
        compiler_params=pltpu.CompilerParams(
            dimension_semantics=("parallel", "arbitrary")),
    )(h_raw, tau, wdb, b_dec2)

    return (recon, h)


# speedup vs baseline: 1.2783x; 1.2783x over previous
import functools
import math

import jax
import jax.numpy as jnp
from jax.experimental import pallas as pl
from jax.experimental.pallas import tpu as pltpu

K_TOP = 64


def _enc_kernel(x_ref, w_ref, b_ref, h_ref):
    acc = jax.lax.dot_general(
        x_ref[...], w_ref[...], (((1,), (1,)), ((), ())),
        preferred_element_type=jnp.float32)
    h_ref[...] = jnp.maximum(acc + b_ref[...].astype(jnp.float32), 0.0)


def _tau_kernel(h_ref, t_ref, *, k_top, n_pass, n_bis):
    tblk, dh = h_ref.shape
    sl = 2048
    ns = dh // sl

    mx = jnp.zeros((tblk, 1), jnp.float32)
    for s in range(ns):
        mx = jnp.maximum(mx, jnp.max(h_ref[:, s * sl:(s + 1) * sl],
                                     axis=1, keepdims=True))

    lo = jnp.zeros((tblk, 1), jnp.float32)
    c_lo = jnp.full((tblk, 1), float(dh), jnp.float32)
    hi = mx
    c_hi = jnp.ones((tblk, 1), jnp.float32)
    log_k = math.log(k_top + 0.5)

    for k in range(n_pass):
        if k < n_pass - n_bis:
            w = (jnp.log(c_lo + 1.0) - log_k) / (
                jnp.log(c_lo + 1.0) - jnp.log(c_hi + 0.5) + 1e-9)
            w = jnp.clip(w, 0.08, 0.92)
            mid = lo + (hi - lo) * w
        else:
            mid = lo + (hi - lo) * 0.5
        cnt = jnp.zeros((tblk, 1), jnp.float32)
        for s in range(ns):
            cnt = cnt + jnp.sum(
                jnp.where(h_ref[:, s * sl:(s + 1) * sl] >= mid, 1.0, 0.0),
                axis=1, keepdims=True)
        take = cnt >= k_top
        lo = jnp.where(take, mid, lo)
        c_lo = jnp.where(take, cnt, c_lo)
        hi = jnp.where(take, hi, mid)
        c_hi = jnp.where(take, c_hi, cnt)

    t_ref[...] = lo


def _dec_kernel(h_ref, t_ref, w_ref, b_ref, o_ref, hm_ref):
    j = pl.program_id(1)
    hm = jnp.where(h_ref[...] >= t_ref[...], h_ref[...], 0.0)
    hm_ref[...] = hm
    acc = jax.lax.dot_general(
        hm.astype(jnp.bfloat16), w_ref[...],
        (((1,), (1,)), ((), ())),
        preferred_element_type=jnp.float32)

    @pl.when(j == 0)
    def _init():
        o_ref[...] = acc + b_ref[...].astype(jnp.float32)

    @pl.when(j > 0)
    def _accum():
        o_ref[...] += acc


def kernel(x, W_enc, b_enc, W_dec, b_dec):
    n_tok, d_in = x.shape
    d_hid = W_enc.shape[0]

    xb = x.astype(jnp.bfloat16)
    web = W_enc.astype(jnp.bfloat16)
    wdb = W_dec.astype(jnp.bfloat16)
    b_enc2 = b_enc.reshape(1, d_hid)
    b_dec2 = b_dec.reshape(1, d_in)

    ta, ha = min(1024, n_tok), 1024
    h_raw = pl.pallas_call(
        _enc_kernel,
        grid=(n_tok // ta, d_hid // ha),
        in_specs=[
            pl.BlockSpec((ta, d_in), lambda i, j: (i, 0)),
            pl.BlockSpec((ha, d_in), lambda i, j: (j, 0)),
            pl.BlockSpec((1, ha), lambda i, j: (0, j)),
        ],
        out_specs=pl.BlockSpec((ta, ha), lambda i, j: (i, j)),
        out_shape=jax.ShapeDtypeStruct((n_tok, d_hid), jnp.float32),
        compiler_params=pltpu.CompilerParams(
            dimension_semantics=("parallel", "arbitrary")),
    )(xb, web, b_enc2)

    tm = min(128, n_tok)
    tau = pl.pallas_call(
        functools.partial(_tau_kernel, k_top=K_TOP, n_pass=16, n_bis=6),
        grid=(n_tok // tm,),
        in_specs=[pl.BlockSpec((tm, d_hid), lambda i: (i, 0))],
        out_specs=pl.BlockSpec((tm, 1), lambda i: (i, 0)),
        out_shape=jax.ShapeDtypeStruct((n_tok, 1), jnp.float32),
        compiler_params=pltpu.CompilerParams(
            dimension_semantics=("parallel",)),
    )(h_raw)

    tb, hb_ = min(512, n_tok), 1024
    recon, h = pl.pallas_call(
        _dec_kernel,
        grid=(n_tok // tb, d_hid // hb_),
        in_specs=[
            pl.BlockSpec((tb, hb_), lambda i, j: (i, j)),
            pl.BlockSpec((tb, 1), lambda i, j: (i, 0)),
            pl.BlockSpec((d_in, hb_), lambda i, j: (0, j)),
            pl.BlockSpec((1, d_in), lambda i, j: (0, 0)),
        ],
        out_specs=(
            pl.BlockSpec((tb, d_in), lambda i, j: (i, 0)),
            pl.BlockSpec((tb, hb_), lambda i, j: (i, j)),
        ),
        out_shape=(
            jax.ShapeDtypeStruct((n_tok, d_in), jnp.float32),
            jax.ShapeDtypeStruct((n_tok, d_hid), jnp.float32),
        ),
        compiler_params=pltpu.CompilerParams(
            dimension_semantics=("parallel", "arbitrary")),
    )(h_raw, tau, wdb, b_dec2)

    return (recon, h)
